# Initial kernel scaffold; baseline (speedup 1.0000x reference)
import jax
import jax.numpy as jnp
from jax.experimental import pallas as pl


def kernel(x, table):
    raise NotImplementedError("write your pallas kernel here")


# speedup vs baseline: 4.2017x; 4.2017x over previous
import functools

import jax
import jax.numpy as jnp
from jax import lax
from jax.experimental import pallas as pl
from jax.experimental.pallas import tpu as pltpu
from jax.experimental.pallas import tpu_sc as plsc

_CHUNK = 128


@functools.lru_cache(maxsize=None)
def _emb_lookup(B: int, D: int):
    info = plsc.get_sparse_core_info()
    NW = info.num_cores * info.num_subcores
    b_per_w = B // NW
    n_chunks = b_per_w // _CHUNK
    assert b_per_w * NW == B and n_chunks * _CHUNK == b_per_w

    mesh = plsc.VectorSubcoreMesh(core_axis_name="c", subcore_axis_name="s")

    @functools.partial(
        pl.kernel,
        mesh=mesh,
        out_type=jax.ShapeDtypeStruct((B, D), jnp.float32),
        scratch_types=[
            pltpu.VMEM((_CHUNK,), jnp.int32),
            pltpu.VMEM((_CHUNK, D), jnp.float32),
            pltpu.SemaphoreType.DMA,
        ],
    )
    def k(idx_hbm, table_hbm, out_hbm, idx_v, rows_v, sem):
        wid = lax.axis_index("s") * info.num_cores + lax.axis_index("c")
        base = wid * b_per_w

        def body(j, carry):
            off = base + j * _CHUNK
            pltpu.sync_copy(idx_hbm.at[pl.ds(off, _CHUNK)], idx_v)
            pltpu.async_copy(table_hbm.at[idx_v], rows_v, sem).wait()
            pltpu.sync_copy(rows_v, out_hbm.at[pl.ds(off, _CHUNK)])
            return carry

        lax.fori_loop(0, n_chunks, body, 0)

    return k


def kernel(x, table):
    S0, S1 = x.shape
    V, D = table.shape
    flat = x.reshape(S0 * S1).astype(jnp.int32)
    out = _emb_lookup(S0 * S1, D)(flat, table)
    return out.reshape(S0, S1, D)


# speedup vs baseline: 5.8377x; 1.3894x over previous
import functools

import jax
import jax.numpy as jnp
from jax import lax
from jax.experimental import pallas as pl
from jax.experimental.pallas import tpu as pltpu
from jax.experimental.pallas import tpu_sc as plsc

_CHUNK = 128


@functools.lru_cache(maxsize=None)
def _emb_lookup(B: int, D: int):
    info = plsc.get_sparse_core_info()
    NW = info.num_cores * info.num_subcores
    b_per_w = B // NW
    n_chunks = b_per_w // _CHUNK
    assert b_per_w * NW == B and n_chunks * _CHUNK == b_per_w
    assert n_chunks % 2 == 0

    mesh = plsc.VectorSubcoreMesh(core_axis_name="c", subcore_axis_name="s")

    @functools.partial(
        pl.kernel,
        mesh=mesh,
        out_type=jax.ShapeDtypeStruct((B, D), jnp.float32),
        scratch_types=[
            pltpu.VMEM((b_per_w,), jnp.int32),
            pltpu.VMEM((_CHUNK, D), jnp.float32),
            pltpu.VMEM((_CHUNK, D), jnp.float32),
            pltpu.SemaphoreType.DMA,
            pltpu.SemaphoreType.DMA,
            pltpu.SemaphoreType.DMA,
            pltpu.SemaphoreType.DMA,
        ],
    )
    def k(idx_hbm, table_hbm, out_hbm, idx_v, rows0, rows1, g0, g1, s0, s1):
        rows = (rows0, rows1)
        gsem = (g0, g1)
        ssem = (s0, s1)
        wid = lax.axis_index("s") * info.num_cores + lax.axis_index("c")
        base = wid * b_per_w

        def gather(j, b):
            pltpu.async_copy(
                table_hbm.at[idx_v.at[pl.ds(j * _CHUNK, _CHUNK)]], rows[b], gsem[b]
            )

        pltpu.sync_copy(idx_hbm.at[pl.ds(base, b_per_w)], idx_v)
        gather(0, 0)

        def body(j0, carry):
            for b in range(2):
                j = j0 * 2 + b
                pltpu.make_async_copy(
                    table_hbm.at[idx_v.at[pl.ds(0, _CHUNK)]], rows[b], gsem[b]
                ).wait()
                pltpu.async_copy(
                    rows[b], out_hbm.at[pl.ds(base + j * _CHUNK, _CHUNK)], ssem[b]
                )
                @pl.when(j > 0)
                def _():
                    pltpu.make_async_copy(
                        rows[1 - b], out_hbm.at[pl.ds(base, _CHUNK)], ssem[1 - b]
                    ).wait()

                jn = jnp.minimum(j + 1, n_chunks - 1)
                gather(jn, 1 - b)
            return carry

        lax.fori_loop(0, n_chunks // 2, body, 0)
        pltpu.make_async_copy(rows[1], out_hbm.at[pl.ds(base, _CHUNK)], ssem[1]).wait()
        pltpu.make_async_copy(
            table_hbm.at[idx_v.at[pl.ds(0, _CHUNK)]], rows[0], gsem[0]
        ).wait()

    return k


def kernel(x, table):
    S0, S1 = x.shape
    V, D = table.shape
    flat = x.reshape(S0 * S1).astype(jnp.int32)
    out = _emb_lookup(S0 * S1, D)(flat, table)
    return out.reshape(S0, S1, D)
